# initial kernel scaffold (unmeasured)
import jax
import jax.numpy as jnp
from jax import lax
from jax.experimental import pallas as pl
from jax.experimental.pallas import tpu as pltpu

N_DEV = 4
SQ = 2048
SKV = 2048
D_MODEL = 1024
H_LOC = 8
DH = 128
BLK = 64
SCALE = 0.08838834764831843


def _compute_body(x_ref, wq_ref, k_ref, v_ref, wo_ref, out_ref):
    h = pl.program_id(0)
    q = jnp.dot(x_ref[...], wq_ref[...], preferred_element_type=jnp.float32)
    k = k_ref[:, 0, :]
    s = lax.dot_general(
        q, k, (((1,), (1,)), ((), ())), preferred_element_type=jnp.float32
    ) * SCALE
    row = lax.broadcasted_iota(jnp.int32, (SQ, SKV), 0)
    col = lax.broadcasted_iota(jnp.int32, (SQ, SKV), 1)
    keep = (col // BLK) <= (row // BLK)
    s = jnp.where(keep, s, -1e9)
    m = jnp.max(s, axis=-1, keepdims=True)
    w = jnp.exp(s - m)
    w = w / jnp.sum(w, axis=-1, keepdims=True)
    ctx = jnp.dot(w, v_ref[:, 0, :], preferred_element_type=jnp.float32)
    contrib = jnp.dot(ctx, wo_ref[...], preferred_element_type=jnp.float32)

    @pl.when(h == 0)
    def _():
        out_ref[...] = contrib

    @pl.when(h != 0)
    def _():
        out_ref[...] += contrib


def _allreduce_body(p_ref, out_ref, comm_ref, send_sems, recv_sems):
    my = lax.axis_index("i")
    left = lax.rem(my + (N_DEV - 1), N_DEV)
    right = lax.rem(my + 1, N_DEV)

    barrier_sem = pltpu.get_barrier_semaphore()
    for nbr in (left, right):
        pl.semaphore_signal(
            barrier_sem, inc=1,
            device_id=(nbr,), device_id_type=pl.DeviceIdType.MESH,
        )
    pl.semaphore_wait(barrier_sem, 2)

    out_ref[...] = p_ref[...]
    comm_ref[0] = p_ref[...]

    for h in range(N_DEV - 1):
        rdma = pltpu.make_async_remote_copy(
            src_ref=comm_ref.at[h],
            dst_ref=comm_ref.at[h + 1],
            send_sem=send_sems.at[h],
            recv_sem=recv_sems.at[h],
            device_id=(right,),
            device_id_type=pl.DeviceIdType.MESH,
        )
        rdma.start()
        rdma.wait()
        out_ref[...] += comm_ref[h + 1]


def kernel(x, Wq, K_ext, V_ext, Wo):
    i = lax.axis_index("i")
    Wq_loc = lax.dynamic_slice(Wq, (0, i * (H_LOC * DH)), (D_MODEL, H_LOC * DH))
    Wo_loc = lax.dynamic_slice(Wo, (i * (H_LOC * DH), 0), (H_LOC * DH, D_MODEL))
    x2 = x.reshape(SQ, D_MODEL)
    K = K_ext.reshape(SKV, H_LOC, DH)
    V = V_ext.reshape(SKV, H_LOC, DH)

    partial = pl.pallas_call(
        _compute_body,
        grid=(H_LOC,),
        in_specs=[
            pl.BlockSpec((SQ, D_MODEL), lambda h: (0, 0)),
            pl.BlockSpec((D_MODEL, DH), lambda h: (0, h)),
            pl.BlockSpec((SKV, 1, DH), lambda h: (0, h, 0)),
            pl.BlockSpec((SKV, 1, DH), lambda h: (0, h, 0)),
            pl.BlockSpec((DH, D_MODEL), lambda h: (h, 0)),
        ],
        out_specs=pl.BlockSpec((SQ, D_MODEL), lambda h: (0, 0)),
        out_shape=jax.ShapeDtypeStruct((SQ, D_MODEL), jnp.float32),
    )(x2, Wq_loc, K, V, Wo_loc)

    out = pl.pallas_call(
        _allreduce_body,
        out_shape=jax.ShapeDtypeStruct((SQ, D_MODEL), jnp.float32),
        in_specs=[pl.BlockSpec(memory_space=pltpu.VMEM)],
        out_specs=pl.BlockSpec(memory_space=pltpu.VMEM),
        scratch_shapes=[
            pltpu.VMEM((N_DEV, SQ, D_MODEL), jnp.float32),
            pltpu.SemaphoreType.DMA((N_DEV - 1,)),
            pltpu.SemaphoreType.DMA((N_DEV - 1,)),
        ],
        compiler_params=pltpu.CompilerParams(collective_id=0),
    )(partial)

    return out.reshape(1, SQ, D_MODEL)


# baseline (device time: 404509 ns/iter reference)
import jax
import jax.numpy as jnp
from jax import lax
from jax.experimental import pallas as pl
from jax.experimental.pallas import tpu as pltpu

N_DEV = 4
SQ = 2048
SKV = 2048
D_MODEL = 1024
H_LOC = 8
DH = 128
BLK = 64
SCALE = 0.08838834764831843


def _compute_body(x_ref, wq_ref, k_ref, v_ref, wo_ref, out_ref):
    h = pl.program_id(0)
    q = jnp.dot(x_ref[...], wq_ref[...], preferred_element_type=jnp.float32)
    k = k_ref[0]
    s = lax.dot_general(
        q, k, (((1,), (1,)), ((), ())), preferred_element_type=jnp.float32
    ) * SCALE
    row = lax.broadcasted_iota(jnp.int32, (SQ, SKV), 0)
    col = lax.broadcasted_iota(jnp.int32, (SQ, SKV), 1)
    keep = (col // BLK) <= (row // BLK)
    s = jnp.where(keep, s, -1e9)
    m = jnp.max(s, axis=-1, keepdims=True)
    w = jnp.exp(s - m)
    w = w / jnp.sum(w, axis=-1, keepdims=True)
    ctx = jnp.dot(w, v_ref[0], preferred_element_type=jnp.float32)
    contrib = jnp.dot(ctx, wo_ref[...], preferred_element_type=jnp.float32)

    @pl.when(h == 0)
    def _():
        out_ref[...] = contrib

    @pl.when(h != 0)
    def _():
        out_ref[...] += contrib


def _allreduce_body(p_ref, out_ref, comm_ref, send_sems, recv_sems):
    my = lax.axis_index("i")
    left = lax.rem(my + (N_DEV - 1), N_DEV)
    right = lax.rem(my + 1, N_DEV)

    barrier_sem = pltpu.get_barrier_semaphore()
    for nbr in (left, right):
        pl.semaphore_signal(
            barrier_sem, inc=1,
            device_id=(nbr,), device_id_type=pl.DeviceIdType.MESH,
        )
    pl.semaphore_wait(barrier_sem, 2)

    out_ref[...] = p_ref[...]
    comm_ref[0] = p_ref[...]

    for h in range(N_DEV - 1):
        rdma = pltpu.make_async_remote_copy(
            src_ref=comm_ref.at[h],
            dst_ref=comm_ref.at[h + 1],
            send_sem=send_sems.at[h],
            recv_sem=recv_sems.at[h],
            device_id=(right,),
            device_id_type=pl.DeviceIdType.MESH,
        )
        rdma.start()
        rdma.wait()
        out_ref[...] += comm_ref[h + 1]


def kernel(x, Wq, K_ext, V_ext, Wo):
    i = lax.axis_index("i")
    Wq_loc = lax.dynamic_slice(Wq, (0, i * (H_LOC * DH)), (D_MODEL, H_LOC * DH))
    Wo_loc = lax.dynamic_slice(Wo, (i * (H_LOC * DH), 0), (H_LOC * DH, D_MODEL))
    x2 = x.reshape(SQ, D_MODEL)
    K = K_ext.reshape(SKV, H_LOC, DH).transpose(1, 0, 2)
    V = V_ext.reshape(SKV, H_LOC, DH).transpose(1, 0, 2)

    partial = pl.pallas_call(
        _compute_body,
        grid=(H_LOC,),
        in_specs=[
            pl.BlockSpec((SQ, D_MODEL), lambda h: (0, 0)),
            pl.BlockSpec((D_MODEL, DH), lambda h: (0, h)),
            pl.BlockSpec((1, SKV, DH), lambda h: (h, 0, 0)),
            pl.BlockSpec((1, SKV, DH), lambda h: (h, 0, 0)),
            pl.BlockSpec((DH, D_MODEL), lambda h: (h, 0)),
        ],
        out_specs=pl.BlockSpec((SQ, D_MODEL), lambda h: (0, 0)),
        out_shape=jax.ShapeDtypeStruct((SQ, D_MODEL), jnp.float32),
    )(x2, Wq_loc, K, V, Wo_loc)

    out = pl.pallas_call(
        _allreduce_body,
        out_shape=jax.ShapeDtypeStruct((SQ, D_MODEL), jnp.float32),
        in_specs=[pl.BlockSpec(memory_space=pltpu.VMEM)],
        out_specs=pl.BlockSpec(memory_space=pltpu.VMEM),
        scratch_shapes=[
            pltpu.VMEM((N_DEV, SQ, D_MODEL), jnp.float32),
            pltpu.SemaphoreType.DMA((N_DEV - 1,)),
            pltpu.SemaphoreType.DMA((N_DEV - 1,)),
        ],
        compiler_params=pltpu.CompilerParams(collective_id=0),
    )(partial)

    return out.reshape(1, SQ, D_MODEL)


# device time: 204397 ns/iter; 1.9790x vs baseline; 1.9790x over previous
import jax
import jax.numpy as jnp
from jax import lax
from jax.experimental import pallas as pl
from jax.experimental.pallas import tpu as pltpu

N_DEV = 4
SQ = 2048
SKV = 2048
D_MODEL = 1024
H_LOC = 8
DH = 128
BLK = 64
SCALE = 0.08838834764831843


def _compute_body(x_ref, wq_ref, k_ref, v_ref, wo_ref, out_ref):
    h = pl.program_id(0)
    q = jnp.dot(x_ref[...], wq_ref[...], preferred_element_type=jnp.float32)
    k = k_ref[0]
    s = lax.dot_general(
        q, k, (((1,), (1,)), ((), ())), preferred_element_type=jnp.float32
    ) * SCALE
    row = lax.broadcasted_iota(jnp.int32, (SQ, SKV), 0)
    col = lax.broadcasted_iota(jnp.int32, (SQ, SKV), 1)
    keep = (col // BLK) <= (row // BLK)
    s = jnp.where(keep, s, -1e9)
    m = jnp.max(s, axis=-1, keepdims=True)
    w = jnp.exp(s - m)
    w = w / jnp.sum(w, axis=-1, keepdims=True)
    ctx = jnp.dot(w, v_ref[0], preferred_element_type=jnp.float32)
    contrib = jnp.dot(ctx, wo_ref[...], preferred_element_type=jnp.float32)

    @pl.when(h == 0)
    def _():
        out_ref[...] = contrib

    @pl.when(h != 0)
    def _():
        out_ref[...] += contrib


CHUNK = SQ // (2 * N_DEV)


def _allreduce_body(p_ref, out_ref, comm_ref, send_sems, recv_sems):
    my = lax.axis_index("i")
    left = lax.rem(my + (N_DEV - 1), N_DEV)
    right = lax.rem(my + 1, N_DEV)

    def mod4(v):
        return lax.rem(v + 4 * N_DEV, N_DEV)

    def rows_r(c):
        return c * CHUNK

    def rows_l(c):
        return N_DEV * CHUNK + c * CHUNK

    barrier_sem = pltpu.get_barrier_semaphore()
    for nbr in (left, right):
        pl.semaphore_signal(
            barrier_sem, inc=1,
            device_id=(nbr,), device_id_type=pl.DeviceIdType.MESH,
        )
    pl.semaphore_wait(barrier_sem, 2)

    out_ref[...] = p_ref[...]

    def copy(src_start, dst_start, dst_is_out, dev, sem_idx):
        dst = out_ref if dst_is_out else comm_ref
        return pltpu.make_async_remote_copy(
            src_ref=out_ref.at[pl.ds(src_start, CHUNK), :],
            dst_ref=dst.at[pl.ds(dst_start, CHUNK), :],
            send_sem=send_sems.at[sem_idx],
            recv_sem=recv_sems.at[sem_idx],
            device_id=(dev,),
            device_id_type=pl.DeviceIdType.MESH,
        )

    for s in range(N_DEV - 1):
        r_send = copy(rows_r(mod4(my - s)), s * CHUNK, False, right, s)
        l_send = copy(rows_l(mod4(my + s)), (3 + s) * CHUNK, False, left, 3 + s)
        r_send.start()
        l_send.start()
        r_send.wait()
        l_send.wait()
        rr = rows_r(mod4(my - s - 1))
        rl = rows_l(mod4(my + s + 1))
        out_ref[pl.ds(rr, CHUNK), :] += comm_ref[pl.ds(s * CHUNK, CHUNK), :]
        out_ref[pl.ds(rl, CHUNK), :] += comm_ref[pl.ds((3 + s) * CHUNK, CHUNK), :]

    for s in range(N_DEV - 1):
        cr = rows_r(mod4(my + 1 - s))
        cl = rows_l(mod4(my - 1 + s))
        r_send = copy(cr, cr, True, right, 6 + s)
        l_send = copy(cl, cl, True, left, 9 + s)
        r_send.start()
        l_send.start()
        r_send.wait()
        l_send.wait()


def kernel(x, Wq, K_ext, V_ext, Wo):
    i = lax.axis_index("i")
    Wq_loc = lax.dynamic_slice(Wq, (0, i * (H_LOC * DH)), (D_MODEL, H_LOC * DH))
    Wo_loc = lax.dynamic_slice(Wo, (i * (H_LOC * DH), 0), (H_LOC * DH, D_MODEL))
    x2 = x.reshape(SQ, D_MODEL)
    K = K_ext.reshape(SKV, H_LOC, DH).transpose(1, 0, 2)
    V = V_ext.reshape(SKV, H_LOC, DH).transpose(1, 0, 2)

    partial = pl.pallas_call(
        _compute_body,
        grid=(H_LOC,),
        in_specs=[
            pl.BlockSpec((SQ, D_MODEL), lambda h: (0, 0)),
            pl.BlockSpec((D_MODEL, DH), lambda h: (0, h)),
            pl.BlockSpec((1, SKV, DH), lambda h: (h, 0, 0)),
            pl.BlockSpec((1, SKV, DH), lambda h: (h, 0, 0)),
            pl.BlockSpec((DH, D_MODEL), lambda h: (h, 0)),
        ],
        out_specs=pl.BlockSpec((SQ, D_MODEL), lambda h: (0, 0)),
        out_shape=jax.ShapeDtypeStruct((SQ, D_MODEL), jnp.float32),
    )(x2, Wq_loc, K, V, Wo_loc)

    out = pl.pallas_call(
        _allreduce_body,
        out_shape=jax.ShapeDtypeStruct((SQ, D_MODEL), jnp.float32),
        in_specs=[pl.BlockSpec(memory_space=pltpu.VMEM)],
        out_specs=pl.BlockSpec(memory_space=pltpu.VMEM),
        scratch_shapes=[
            pltpu.VMEM((6 * CHUNK, D_MODEL), jnp.float32),
            pltpu.SemaphoreType.DMA((12,)),
            pltpu.SemaphoreType.DMA((12,)),
        ],
        compiler_params=pltpu.CompilerParams(collective_id=0),
    )(partial)

    return out.reshape(1, SQ, D_MODEL)
